# Initial kernel scaffold; baseline (speedup 1.0000x reference)
import functools

import jax
import jax.numpy as jnp
from jax import lax
from jax.experimental import pallas as pl
from jax.experimental.pallas import tpu as pltpu
from jax.experimental.pallas import tpu_sc as plsc

NC = 2
NS = 16
NW = NC * NS
B = 80
DEG_W = 16


def _make_agg(N, D, NB):
    mesh = plsc.VectorSubcoreMesh(core_axis_name="c", subcore_axis_name="s")
    RPT = N // NS
    ZR = 125

    @functools.partial(
        pl.kernel,
        out_type=jax.ShapeDtypeStruct((NC, N, D), jnp.float32),
        mesh=mesh,
        scratch_types=[
            pltpu.VMEM((NB, B), jnp.int32),
            pltpu.VMEM((NB, B), jnp.int32),
            pltpu.VMEM((B, D), jnp.float32),
            pltpu.VMEM((B, D), jnp.float32),
            pltpu.VMEM((125, D), jnp.float32),
            pltpu.VMEM_SHARED((N, D), jnp.float32),
            pltpu.SemaphoreType.DMA,
            pltpu.SemaphoreType.DMA,
        ],
    )
    def agg(rows_hbm, src_hbm, dst_hbm, zz_hbm, out_hbm,
            src_v, dst_v, buf0, buf1, zbuf, acc_sh, sem0, sem1):
        ZRl = 125
        cid = lax.axis_index("c")
        sid = lax.axis_index("s")
        wid = cid * NS + sid

        pltpu.sync_copy(zz_hbm, zbuf)

        @pl.loop(0, RPT // ZRl)
        def _(k):
            pltpu.sync_copy(zbuf, acc_sh.at[pl.ds(sid * RPT + k * ZRl, ZRl)])

        pltpu.sync_copy(src_hbm.at[wid], src_v)
        pltpu.sync_copy(dst_hbm.at[wid], dst_v)
        plsc.subcore_barrier()

        pltpu.async_copy(rows_hbm.at[src_v.at[0]], buf0, sem0).wait()
        pltpu.sync_copy(buf0, acc_sh.at[dst_v.at[0]], add=True)

        @pl.loop(1, NB, step=2)
        def _(g):
            c0 = pltpu.async_copy(rows_hbm.at[src_v.at[g]], buf0, sem0)
            c1 = pltpu.async_copy(rows_hbm.at[src_v.at[g + 1]], buf1, sem1)
            c0.wait()
            pltpu.sync_copy(buf0, acc_sh.at[dst_v.at[g]], add=True)
            c1.wait()
            pltpu.sync_copy(buf1, acc_sh.at[dst_v.at[g + 1]], add=True)

        plsc.subcore_barrier()

        @pl.loop(0, RPT // ZRl)
        def _(k):
            r0 = sid * RPT + k * ZRl
            pltpu.sync_copy(acc_sh.at[pl.ds(r0, ZRl)], zbuf)
            pltpu.sync_copy(zbuf, out_hbm.at[cid, pl.ds(r0, ZRl)])

    return agg


def _make_deg(N, NB):
    mesh = plsc.VectorSubcoreMesh(core_axis_name="c", subcore_axis_name="s")
    RPT = N // NS

    @functools.partial(
        pl.kernel,
        out_type=jax.ShapeDtypeStruct((NC, N, DEG_W), jnp.float32),
        mesh=mesh,
        scratch_types=[
            pltpu.VMEM((NB, B), jnp.int32),
            pltpu.VMEM((B, DEG_W), jnp.float32),
            pltpu.VMEM((N // NS, DEG_W), jnp.float32),
            pltpu.VMEM_SHARED((N, DEG_W), jnp.float32),
        ],
    )
    def deg(dst_hbm, ones_hbm, zz_hbm, out_hbm, dst_v, ones_v, zbuf, acc_sh):
        cid = lax.axis_index("c")
        sid = lax.axis_index("s")
        wid = cid * NS + sid

        pltpu.sync_copy(zz_hbm, zbuf)
        pltpu.sync_copy(zbuf, acc_sh.at[pl.ds(sid * RPT, RPT)])
        pltpu.sync_copy(ones_hbm, ones_v)
        pltpu.sync_copy(dst_hbm.at[wid], dst_v)
        plsc.subcore_barrier()

        @pl.loop(0, NB)
        def _(g):
            pltpu.sync_copy(ones_v, acc_sh.at[dst_v.at[g]], add=True)

        plsc.subcore_barrier()
        pltpu.sync_copy(acc_sh.at[pl.ds(sid * RPT, RPT)], zbuf)
        pltpu.sync_copy(zbuf, out_hbm.at[cid, pl.ds(sid * RPT, RPT)])

    return deg


def _dis_from(d0, d1):
    deg = d0 + d1
    return jnp.where(deg > 0, lax.rsqrt(jnp.maximum(deg, 1.0)), 0.0)


def _mm_scale(x, W, d0, d1, R=2000):
    N, D = x.shape

    def body(x_ref, w_ref, d0_ref, d1_ref, hs_ref, dis_ref):
        dis = _dis_from(d0_ref[...], d1_ref[...])
        h = jnp.dot(x_ref[...], w_ref[...], preferred_element_type=jnp.float32)
        hs_ref[...] = h * dis
        dis_ref[...] = dis

    return pl.pallas_call(
        body,
        grid=(N // R,),
        in_specs=[
            pl.BlockSpec((R, D), lambda i: (i, 0)),
            pl.BlockSpec((D, D), lambda i: (0, 0)),
            pl.BlockSpec((R, 1), lambda i: (i, 0)),
            pl.BlockSpec((R, 1), lambda i: (i, 0)),
        ],
        out_specs=[
            pl.BlockSpec((R, D), lambda i: (i, 0)),
            pl.BlockSpec((R, 1), lambda i: (i, 0)),
        ],
        out_shape=[
            jax.ShapeDtypeStruct((N, D), jnp.float32),
            jax.ShapeDtypeStruct((N, 1), jnp.float32),
        ],
    )(x, W, d0, d1)


def _layer2(p0, p1, dis, b1, W2, R=2000):
    N, D = p0.shape

    def body(p0_ref, p1_ref, dis_ref, b1_ref, w_ref, o_ref):
        dis = dis_ref[...]
        t = jax.nn.relu((p0_ref[...] + p1_ref[...]) * dis + b1_ref[...])
        o_ref[...] = jnp.dot(t, w_ref[...],
                             preferred_element_type=jnp.float32) * dis

    return pl.pallas_call(
        body,
        grid=(N // R,),
        in_specs=[
            pl.BlockSpec((R, D), lambda i: (i, 0)),
            pl.BlockSpec((R, D), lambda i: (i, 0)),
            pl.BlockSpec((R, 1), lambda i: (i, 0)),
            pl.BlockSpec((1, D), lambda i: (0, 0)),
            pl.BlockSpec((D, D), lambda i: (0, 0)),
        ],
        out_specs=pl.BlockSpec((R, D), lambda i: (i, 0)),
        out_shape=jax.ShapeDtypeStruct((N, D), jnp.float32),
    )(p0, p1, dis, b1, W2)


def _final(q0, q1, dis, b2, R=2000):
    N, D = q0.shape

    def body(q0_ref, q1_ref, dis_ref, b2_ref, o_ref):
        o_ref[...] = (q0_ref[...] + q1_ref[...]) * dis_ref[...] + b2_ref[...]

    return pl.pallas_call(
        body,
        grid=(N // R,),
        in_specs=[
            pl.BlockSpec((R, D), lambda i: (i, 0)),
            pl.BlockSpec((R, D), lambda i: (i, 0)),
            pl.BlockSpec((R, 1), lambda i: (i, 0)),
            pl.BlockSpec((1, D), lambda i: (0, 0)),
        ],
        out_specs=pl.BlockSpec((R, D), lambda i: (i, 0)),
        out_shape=jax.ShapeDtypeStruct((N, D), jnp.float32),
    )(q0, q1, dis, b2)


def kernel(x, edge_index, W1, b1, W2, b2):
    N, D = x.shape
    E = edge_index.shape[1]
    NB = E // (NW * B)

    srcr = edge_index[0].reshape(NW, NB, B)
    dstr = edge_index[1].reshape(NW, NB, B)
    ones = jnp.ones((B, DEG_W), jnp.float32)
    zdeg = jnp.zeros((N // NS, DEG_W), jnp.float32)
    zz = jnp.zeros((125, D), jnp.float32)

    deg_p = _make_deg(N, NB)(dstr, ones, zdeg)
    d0 = deg_p[0, :, 0:1]
    d1 = deg_p[1, :, 0:1]

    hs1, dis = _mm_scale(x, W1, d0, d1)
    p = _make_agg(N, D, NB)(hs1, srcr, dstr, zz)
    hs2 = _layer2(p[0], p[1], dis, b1.reshape(1, D), W2)
    q = _make_agg(N, D, NB)(hs2, srcr, dstr, zz)
    out = _final(q[0], q[1], dis, b2.reshape(1, D))
    return out


# speedup vs baseline: 10.0414x; 10.0414x over previous
import functools

import jax
import jax.numpy as jnp
from jax import lax
from jax.experimental import pallas as pl
from jax.experimental.pallas import tpu as pltpu
from jax.experimental.pallas import tpu_sc as plsc

NC = 2
NS = 16
NW = NC * NS
B = 40
DEG_W = 128


def _make_agg(N, N_pad, D, E):
    EPT = E // NW
    NB = EPT // B
    mesh = plsc.VectorSubcoreMesh(core_axis_name="c", subcore_axis_name="s")
    RPT = N_pad // NS

    @functools.partial(
        pl.kernel,
        out_type=jax.ShapeDtypeStruct((NC, N_pad, D), jnp.float32),
        mesh=mesh,
        scratch_types=[
            pltpu.VMEM((B,), jnp.int32),
            pltpu.VMEM((B,), jnp.int32),
            pltpu.VMEM((B,), jnp.int32),
            pltpu.VMEM((B,), jnp.int32),
            pltpu.VMEM((B, D), jnp.float32),
            pltpu.VMEM((B, D), jnp.float32),
            pltpu.VMEM_SHARED((N_pad, D), jnp.float32),
            pltpu.SemaphoreType.DMA,
            pltpu.SemaphoreType.DMA,
        ],
    )
    def agg(rows_hbm, src_hbm, dst_hbm, zz_hbm, out_hbm,
            src0, src1, dst0, dst1, buf0, buf1, acc_sh, sem0, sem1):
        cid = lax.axis_index("c")
        sid = lax.axis_index("s")
        base = (cid * NS + sid) * EPT

        pltpu.sync_copy(zz_hbm, acc_sh.at[pl.ds(sid * RPT, RPT)])
        plsc.subcore_barrier()

        @pl.loop(0, NB, step=2)
        def _(g):
            pltpu.sync_copy(src_hbm.at[pl.ds(base + g * B, B)], src0)
            c0 = pltpu.async_copy(rows_hbm.at[src0], buf0, sem0)
            pltpu.sync_copy(src_hbm.at[pl.ds(base + (g + 1) * B, B)], src1)
            c1 = pltpu.async_copy(rows_hbm.at[src1], buf1, sem1)
            pltpu.sync_copy(dst_hbm.at[pl.ds(base + g * B, B)], dst0)
            pltpu.sync_copy(dst_hbm.at[pl.ds(base + (g + 1) * B, B)], dst1)
            c0.wait()
            pltpu.sync_copy(buf0, acc_sh.at[dst0], add=True)
            c1.wait()
            pltpu.sync_copy(buf1, acc_sh.at[dst1], add=True)

        plsc.subcore_barrier()

        pltpu.sync_copy(acc_sh.at[pl.ds(sid * RPT, RPT)],
                        out_hbm.at[cid, pl.ds(sid * RPT, RPT)])

    return agg


def _make_deg(N_pad, E):
    EPT = E // NW
    NB = EPT // B
    mesh = plsc.VectorSubcoreMesh(core_axis_name="c", subcore_axis_name="s")
    RPT = N_pad // NS

    @functools.partial(
        pl.kernel,
        out_type=jax.ShapeDtypeStruct((NC, N_pad, DEG_W), jnp.float32),
        mesh=mesh,
        scratch_types=[
            pltpu.VMEM((B,), jnp.int32),
            pltpu.VMEM((B, DEG_W), jnp.float32),
            pltpu.VMEM((N_pad // NS, DEG_W), jnp.float32),
            pltpu.VMEM_SHARED((N_pad, DEG_W), jnp.float32),
        ],
    )
    def deg(dst_hbm, ones_hbm, zz_hbm, out_hbm, dst_v, ones_v, zbuf, acc_sh):
        cid = lax.axis_index("c")
        sid = lax.axis_index("s")

        base = (cid * NS + sid) * EPT

        pltpu.sync_copy(zz_hbm, acc_sh.at[pl.ds(sid * RPT, RPT)])
        pltpu.sync_copy(ones_hbm, ones_v)
        plsc.subcore_barrier()

        @pl.loop(0, NB)
        def _(g):
            pltpu.sync_copy(dst_hbm.at[pl.ds(base + g * B, B)], dst_v)
            pltpu.sync_copy(ones_v, acc_sh.at[dst_v], add=True)

        plsc.subcore_barrier()
        pltpu.sync_copy(acc_sh.at[pl.ds(sid * RPT, RPT)],
                        out_hbm.at[cid, pl.ds(sid * RPT, RPT)])

    return deg


def _dis_from(d0, d1):
    deg = d0 + d1
    return jnp.where(deg > 0, lax.rsqrt(jnp.maximum(deg, 1.0)), 0.0)


def _mm_scale(x, W, d0, d1, R=2000):
    N, D = x.shape

    def body(x_ref, w_ref, d0_ref, d1_ref, hs_ref, dis_ref):
        dis = _dis_from(d0_ref[...], d1_ref[...])
        h = jnp.dot(x_ref[...], w_ref[...], preferred_element_type=jnp.float32)
        hs_ref[...] = h * dis
        dis_ref[...] = dis

    return pl.pallas_call(
        body,
        grid=(N // R,),
        in_specs=[
            pl.BlockSpec((R, D), lambda i: (i, 0)),
            pl.BlockSpec((D, D), lambda i: (0, 0)),
            pl.BlockSpec((R, 1), lambda i: (i, 0)),
            pl.BlockSpec((R, 1), lambda i: (i, 0)),
        ],
        out_specs=[
            pl.BlockSpec((R, D), lambda i: (i, 0)),
            pl.BlockSpec((R, 1), lambda i: (i, 0)),
        ],
        out_shape=[
            jax.ShapeDtypeStruct((N, D), jnp.float32),
            jax.ShapeDtypeStruct((N, 1), jnp.float32),
        ],
    )(x, W, d0, d1)


def _layer2(p0, p1, dis, b1, W2, R=2000):
    N, D = p0.shape

    def body(p0_ref, p1_ref, dis_ref, b1_ref, w_ref, o_ref):
        dis = dis_ref[...]
        t = jax.nn.relu((p0_ref[...] + p1_ref[...]) * dis + b1_ref[...])
        o_ref[...] = jnp.dot(t, w_ref[...],
                             preferred_element_type=jnp.float32) * dis

    return pl.pallas_call(
        body,
        grid=(N // R,),
        in_specs=[
            pl.BlockSpec((R, D), lambda i: (i, 0)),
            pl.BlockSpec((R, D), lambda i: (i, 0)),
            pl.BlockSpec((R, 1), lambda i: (i, 0)),
            pl.BlockSpec((1, D), lambda i: (0, 0)),
            pl.BlockSpec((D, D), lambda i: (0, 0)),
        ],
        out_specs=pl.BlockSpec((R, D), lambda i: (i, 0)),
        out_shape=jax.ShapeDtypeStruct((N, D), jnp.float32),
    )(p0, p1, dis, b1, W2)


def _final(q0, q1, dis, b2, R=2000):
    N, D = q0.shape

    def body(q0_ref, q1_ref, dis_ref, b2_ref, o_ref):
        o_ref[...] = (q0_ref[...] + q1_ref[...]) * dis_ref[...] + b2_ref[...]

    return pl.pallas_call(
        body,
        grid=(N // R,),
        in_specs=[
            pl.BlockSpec((R, D), lambda i: (i, 0)),
            pl.BlockSpec((R, D), lambda i: (i, 0)),
            pl.BlockSpec((R, 1), lambda i: (i, 0)),
            pl.BlockSpec((1, D), lambda i: (0, 0)),
        ],
        out_specs=pl.BlockSpec((R, D), lambda i: (i, 0)),
        out_shape=jax.ShapeDtypeStruct((N, D), jnp.float32),
    )(q0, q1, dis, b2)


def kernel(x, edge_index, W1, b1, W2, b2):
    N, D = x.shape
    E = edge_index.shape[1]
    N_pad = ((N + 8 * NS - 1) // (8 * NS)) * (8 * NS)

    src = edge_index[0]
    dst = edge_index[1]
    ones = jnp.ones((B, DEG_W), jnp.float32)
    zdeg = jnp.zeros((N_pad // NS, DEG_W), jnp.float32)
    zz = jnp.zeros((N_pad // NS, D), jnp.float32)

    deg_p = _make_deg(N_pad, E)(dst, ones, zdeg)
    d0 = deg_p[0, :N, 0:1]
    d1 = deg_p[1, :N, 0:1]

    hs1, dis = _mm_scale(x, W1, d0, d1)
    p = _make_agg(N, N_pad, D, E)(hs1, src, dst, zz)
    hs2 = _layer2(p[0, :N], p[1, :N], dis, b1.reshape(1, D), W2)
    q = _make_agg(N, N_pad, D, E)(hs2, src, dst, zz)
    out = _final(q[0, :N], q[1, :N], dis, b2.reshape(1, D))
    return out


# speedup vs baseline: 17.7470x; 1.7674x over previous
import functools

import jax
import jax.numpy as jnp
from jax import lax
from jax.experimental import pallas as pl
from jax.experimental.pallas import tpu as pltpu
from jax.experimental.pallas import tpu_sc as plsc

NC = 2
NS = 16
NW = NC * NS
B = 80
DEG_W = 128


def _make_agg(N, N_pad, D, E):
    EPT = E // NW
    NB = EPT // B
    mesh = plsc.VectorSubcoreMesh(core_axis_name="c", subcore_axis_name="s")
    RPT = N_pad // NS

    @functools.partial(
        pl.kernel,
        out_type=jax.ShapeDtypeStruct((NC, N_pad, D), jnp.float32),
        mesh=mesh,
        scratch_types=[
            pltpu.VMEM((EPT,), jnp.int32),
            pltpu.VMEM((NB, B), jnp.int32),
            pltpu.VMEM((B, D), jnp.float32),
            pltpu.VMEM((B, D), jnp.float32),
            pltpu.VMEM_SHARED((N_pad, D), jnp.float32),
            pltpu.SemaphoreType.DMA,
            pltpu.SemaphoreType.DMA,
        ],
    )
    def agg(rows_hbm, src_hbm, dst_hbm, zz_hbm, out_hbm,
            src_v, dst_v, buf0, buf1, acc_sh, sem0, sem1):
        cid = lax.axis_index("c")
        sid = lax.axis_index("s")
        wid = cid * NS + sid
        base = wid * EPT

        pltpu.sync_copy(zz_hbm, acc_sh.at[pl.ds(sid * RPT, RPT)])
        pltpu.sync_copy(src_hbm.at[pl.ds(base, EPT)], src_v)
        pltpu.sync_copy(dst_hbm.at[wid], dst_v)
        plsc.subcore_barrier()

        def gather(g, buf, sem):
            return pltpu.async_copy(
                rows_hbm.at[src_v.at[pl.ds(g * B, B)]], buf, sem)

        def scatter(g, buf):
            pltpu.sync_copy(buf, acc_sh.at[dst_v.at[g]], add=True)

        gather(0, buf0, sem0).wait()
        scatter(0, buf0)

        @pl.loop(1, NB, step=2)
        def _(g):
            c0 = gather(g, buf0, sem0)
            c1 = gather(g + 1, buf1, sem1)
            c0.wait()
            scatter(g, buf0)
            c1.wait()
            scatter(g + 1, buf1)

        plsc.subcore_barrier()

        pltpu.sync_copy(acc_sh.at[pl.ds(sid * RPT, RPT)],
                        out_hbm.at[cid, pl.ds(sid * RPT, RPT)])

    return agg


def _make_deg(N_pad, E):
    EPT = E // NW
    NB = EPT // B
    mesh = plsc.VectorSubcoreMesh(core_axis_name="c", subcore_axis_name="s")
    RPT = N_pad // NS

    @functools.partial(
        pl.kernel,
        out_type=jax.ShapeDtypeStruct((NC, N_pad, DEG_W), jnp.float32),
        mesh=mesh,
        scratch_types=[
            pltpu.VMEM((NB, B), jnp.int32),
            pltpu.VMEM((B, DEG_W), jnp.float32),
            pltpu.VMEM_SHARED((N_pad, DEG_W), jnp.float32),
            pltpu.SemaphoreType.DMA,
            pltpu.SemaphoreType.DMA,
        ],
    )
    def deg(dst_hbm, ones_hbm, zz_hbm, out_hbm, dst_v, ones_v, acc_sh,
            sem0, sem1):
        cid = lax.axis_index("c")
        sid = lax.axis_index("s")
        wid = cid * NS + sid

        pltpu.sync_copy(zz_hbm, acc_sh.at[pl.ds(sid * RPT, RPT)])
        pltpu.sync_copy(ones_hbm, ones_v)
        pltpu.sync_copy(dst_hbm.at[wid], dst_v)
        plsc.subcore_barrier()

        pltpu.async_copy(ones_v, acc_sh.at[dst_v.at[0]], sem0,
                         add=True).wait()

        @pl.loop(1, NB, step=2)
        def _(g):
            c0 = pltpu.async_copy(ones_v, acc_sh.at[dst_v.at[g]], sem0,
                                  add=True)
            c1 = pltpu.async_copy(ones_v, acc_sh.at[dst_v.at[g + 1]], sem1,
                                  add=True)
            c0.wait()
            c1.wait()

        plsc.subcore_barrier()
        pltpu.sync_copy(acc_sh.at[pl.ds(sid * RPT, RPT)],
                        out_hbm.at[cid, pl.ds(sid * RPT, RPT)])

    return deg


def _dis_from(d0, d1):
    deg = d0 + d1
    return jnp.where(deg > 0, lax.rsqrt(jnp.maximum(deg, 1.0)), 0.0)


def _mm_scale(x, W, d0, d1, R=2000):
    N, D = x.shape

    def body(x_ref, w_ref, d0_ref, d1_ref, hs_ref, dis_ref):
        dis = _dis_from(d0_ref[...], d1_ref[...])
        h = jnp.dot(x_ref[...], w_ref[...], preferred_element_type=jnp.float32)
        hs_ref[...] = h * dis
        dis_ref[...] = dis

    return pl.pallas_call(
        body,
        grid=(N // R,),
        in_specs=[
            pl.BlockSpec((R, D), lambda i: (i, 0)),
            pl.BlockSpec((D, D), lambda i: (0, 0)),
            pl.BlockSpec((R, 1), lambda i: (i, 0)),
            pl.BlockSpec((R, 1), lambda i: (i, 0)),
        ],
        out_specs=[
            pl.BlockSpec((R, D), lambda i: (i, 0)),
            pl.BlockSpec((R, 1), lambda i: (i, 0)),
        ],
        out_shape=[
            jax.ShapeDtypeStruct((N, D), jnp.float32),
            jax.ShapeDtypeStruct((N, 1), jnp.float32),
        ],
    )(x, W, d0, d1)


def _layer2(p0, p1, dis, b1, W2, R=2000):
    N, D = p0.shape

    def body(p0_ref, p1_ref, dis_ref, b1_ref, w_ref, o_ref):
        dis = dis_ref[...]
        t = jax.nn.relu((p0_ref[...] + p1_ref[...]) * dis + b1_ref[...])
        o_ref[...] = jnp.dot(t, w_ref[...],
                             preferred_element_type=jnp.float32) * dis

    return pl.pallas_call(
        body,
        grid=(N // R,),
        in_specs=[
            pl.BlockSpec((R, D), lambda i: (i, 0)),
            pl.BlockSpec((R, D), lambda i: (i, 0)),
            pl.BlockSpec((R, 1), lambda i: (i, 0)),
            pl.BlockSpec((1, D), lambda i: (0, 0)),
            pl.BlockSpec((D, D), lambda i: (0, 0)),
        ],
        out_specs=pl.BlockSpec((R, D), lambda i: (i, 0)),
        out_shape=jax.ShapeDtypeStruct((N, D), jnp.float32),
    )(p0, p1, dis, b1, W2)


def _final(q0, q1, dis, b2, R=2000):
    N, D = q0.shape

    def body(q0_ref, q1_ref, dis_ref, b2_ref, o_ref):
        o_ref[...] = (q0_ref[...] + q1_ref[...]) * dis_ref[...] + b2_ref[...]

    return pl.pallas_call(
        body,
        grid=(N // R,),
        in_specs=[
            pl.BlockSpec((R, D), lambda i: (i, 0)),
            pl.BlockSpec((R, D), lambda i: (i, 0)),
            pl.BlockSpec((R, 1), lambda i: (i, 0)),
            pl.BlockSpec((1, D), lambda i: (0, 0)),
        ],
        out_specs=pl.BlockSpec((R, D), lambda i: (i, 0)),
        out_shape=jax.ShapeDtypeStruct((N, D), jnp.float32),
    )(q0, q1, dis, b2)


def kernel(x, edge_index, W1, b1, W2, b2):
    N, D = x.shape
    E = edge_index.shape[1]
    N_pad = ((N + 8 * NS - 1) // (8 * NS)) * (8 * NS)

    EPT = E // NW
    src = edge_index[0]
    dstr = edge_index[1].reshape(NW, EPT // B, B)
    ones = jnp.ones((B, DEG_W), jnp.float32)
    zdeg = jnp.zeros((N_pad // NS, DEG_W), jnp.float32)
    zz = jnp.zeros((N_pad // NS, D), jnp.float32)

    deg_p = _make_deg(N_pad, E)(dstr, ones, zdeg)
    d0 = deg_p[0, :N, 0:1]
    d1 = deg_p[1, :N, 0:1]

    hs1, dis = _mm_scale(x, W1, d0, d1)
    p = _make_agg(N, N_pad, D, E)(hs1, src, dstr, zz)
    hs2 = _layer2(p[0, :N], p[1, :N], dis, b1.reshape(1, D), W2)
    q = _make_agg(N, N_pad, D, E)(hs2, src, dstr, zz)
    out = _final(q[0, :N], q[1, :N], dis, b2.reshape(1, D))
    return out
